# Initial kernel scaffold; baseline (speedup 1.0000x reference)
import jax
import jax.numpy as jnp
from jax.experimental import pallas as pl


def kernel(x, enc_mask_token, token_nodes, noise_nodes, noise_src, mask_nodes):
    raise NotImplementedError("write your pallas kernel here")


# speedup vs baseline: 3.8635x; 3.8635x over previous
import functools

import jax
import jax.numpy as jnp
from jax import lax
from jax.experimental import pallas as pl
from jax.experimental.pallas import tpu as pltpu
from jax.experimental.pallas import tpu_sc as plsc

_NC = 2
_NS = 16
_NW = _NC * _NS


def _masked_copy_body(x_ref, f_ref, m_ref, o_ref):
    o_ref[...] = jnp.where(f_ref[...] > 0, m_ref[...], x_ref[...])


def _masked_copy(x, flag, mask_token, rows_per_block):
    n, d = x.shape
    grid = pl.cdiv(n, rows_per_block)
    return pl.pallas_call(
        _masked_copy_body,
        grid=(grid,),
        in_specs=[
            pl.BlockSpec((rows_per_block, d), lambda g: (g, 0)),
            pl.BlockSpec((rows_per_block, 1), lambda g: (g, 0)),
            pl.BlockSpec((1, d), lambda g: (0, 0)),
        ],
        out_specs=pl.BlockSpec((rows_per_block, d), lambda g: (g, 0)),
        out_shape=jax.ShapeDtypeStruct((n, d), x.dtype),
        compiler_params=pltpu.CompilerParams(
            dimension_semantics=("arbitrary",),
        ),
    )(x, flag, mask_token)


def _make_noise_scatter(n, d, chunk):
    mesh = plsc.VectorSubcoreMesh(core_axis_name="c", subcore_axis_name="s")

    @functools.partial(
        pl.kernel,
        out_type=(),
        mesh=mesh,
        scratch_types=[
            pltpu.VMEM((chunk,), jnp.int32),
            pltpu.VMEM((chunk,), jnp.int32),
            pltpu.VMEM((chunk, d), jnp.float32),
            pltpu.SemaphoreType.DMA,
        ],
    )
    def noise_scatter(x_hbm, src_hbm, dst_hbm, out_ref, src_v, dst_v, rows_v, sem):
        wid = lax.axis_index("s") * _NC + lax.axis_index("c")
        base = wid * chunk
        pltpu.sync_copy(src_hbm.at[pl.ds(base, chunk)], src_v)
        pltpu.sync_copy(dst_hbm.at[pl.ds(base, chunk)], dst_v)
        pltpu.async_copy(x_hbm.at[src_v], rows_v, sem).wait()
        pltpu.async_copy(rows_v, out_ref.at[dst_v], sem).wait()

    return noise_scatter


def kernel(x, enc_mask_token, token_nodes, noise_nodes, noise_src, mask_nodes):
    n, d = x.shape
    k = noise_nodes.shape[0]

    flag = (
        jnp.zeros((n, 1), jnp.int32)
        .at[token_nodes]
        .set(1, unique_indices=True, mode="promise_in_bounds")
    )

    out = _masked_copy(x, flag, enc_mask_token, rows_per_block=1000)

    chunk = ((k + _NW - 1) // _NW + 7) // 8 * 8
    kp = chunk * _NW
    src_p = jnp.concatenate([noise_src, jnp.broadcast_to(noise_src[:1], (kp - k,))])
    dst_p = jnp.concatenate([noise_nodes, jnp.broadcast_to(noise_nodes[:1], (kp - k,))])

    out_ref = jax.new_ref(out)
    _make_noise_scatter(n, d, chunk)(x, src_p, dst_p, out_ref)
    return jax.freeze(out_ref)


# speedup vs baseline: 3.9920x; 1.0333x over previous
import functools

import jax
import jax.numpy as jnp
from jax import lax
from jax.experimental import pallas as pl
from jax.experimental.pallas import tpu as pltpu
from jax.experimental.pallas import tpu_sc as plsc

_NC = 2
_NS = 16
_NW = _NC * _NS


def _masked_copy_body(x_ref, f_ref, m_ref, o_ref):
    o_ref[...] = jnp.where(f_ref[...] > 0, m_ref[...], x_ref[...])


def _masked_copy(x, flag, mask_token, rows_per_block):
    n, d = x.shape
    grid = pl.cdiv(n, rows_per_block)
    return pl.pallas_call(
        _masked_copy_body,
        grid=(grid,),
        in_specs=[
            pl.BlockSpec((rows_per_block, d), lambda g: (g, 0)),
            pl.BlockSpec((rows_per_block, 1), lambda g: (g, 0)),
            pl.BlockSpec((1, d), lambda g: (0, 0)),
        ],
        out_specs=pl.BlockSpec((rows_per_block, d), lambda g: (g, 0)),
        out_shape=jax.ShapeDtypeStruct((n, d), x.dtype),
        compiler_params=pltpu.CompilerParams(
            dimension_semantics=("arbitrary",),
        ),
    )(x, flag, mask_token)


def _make_noise_scatter(n, d, chunk):
    mesh = plsc.VectorSubcoreMesh(core_axis_name="c", subcore_axis_name="s")

    @functools.partial(
        pl.kernel,
        out_type=(),
        mesh=mesh,
        scratch_types=[
            pltpu.VMEM((chunk,), jnp.int32),
            pltpu.VMEM((chunk,), jnp.int32),
            pltpu.VMEM((chunk, d), jnp.float32),
            pltpu.SemaphoreType.DMA,
        ],
    )
    def noise_scatter(x_hbm, src_hbm, dst_hbm, out_ref, src_v, dst_v, rows_v, sem):
        wid = lax.axis_index("s") * _NC + lax.axis_index("c")
        base = wid * chunk
        pltpu.sync_copy(src_hbm.at[pl.ds(base, chunk)], src_v)
        pltpu.sync_copy(dst_hbm.at[pl.ds(base, chunk)], dst_v)
        pltpu.async_copy(x_hbm.at[src_v], rows_v, sem).wait()
        pltpu.async_copy(rows_v, out_ref.at[dst_v], sem).wait()

    return noise_scatter


def kernel(x, enc_mask_token, token_nodes, noise_nodes, noise_src, mask_nodes):
    n, d = x.shape
    k = noise_nodes.shape[0]

    flag = (
        jnp.zeros((n, 1), jnp.int32)
        .at[token_nodes]
        .set(1, unique_indices=True, mode="promise_in_bounds")
    )

    out = _masked_copy(x, flag, enc_mask_token, rows_per_block=2000)

    chunk = ((k + _NW - 1) // _NW + 7) // 8 * 8
    kp = chunk * _NW
    src_p = jnp.concatenate([noise_src, jnp.broadcast_to(noise_src[:1], (kp - k,))])
    dst_p = jnp.concatenate([noise_nodes, jnp.broadcast_to(noise_nodes[:1], (kp - k,))])

    out_ref = jax.new_ref(out)
    _make_noise_scatter(n, d, chunk)(x, src_p, dst_p, out_ref)
    return jax.freeze(out_ref)
